# baseline (speedup 1.0000x reference)
import functools

import jax
import jax.numpy as jnp
from jax import lax
from jax.experimental import pallas as pl
from jax.experimental.pallas import tpu as pltpu
from jax.experimental.pallas import tpu_sc as plsc

_N = 50000
_F = 20
_V = 100000
_D = 64
_E = 800000

_NP = 50176
_NPT = _NP // 32
_NPS = _NP // 16
_BCH = 32
_BST = _BCH * _F // 128
_BCHUNKS = _NPT // _BCH
_FIROWS = _NP * _F // 128
_EP = 802816
_EROWS = _EP // 128
_ERPT = _EROWS // 16
_ECH = 8
_ECHUNKS = _ERPT // _ECH

_mesh = plsc.VectorSubcoreMesh(core_axis_name="c", subcore_axis_name="s")


@functools.partial(
    pl.kernel,
    mesh=_mesh,
    out_type=(
        jax.ShapeDtypeStruct((_NP,), jnp.float32),
        jax.ShapeDtypeStruct((_NP,), jnp.float32),
    ),
    compiler_params=pltpu.CompilerParams(use_tc_tiling_on_sc=False,
                                         needs_layout_passes=False),
    scratch_types=(
        pltpu.VMEM((_ECH, 128), jnp.int32),
        pltpu.VMEM((_ECH, 128), jnp.int32),
        pltpu.VMEM((128,), jnp.float32),
        pltpu.VMEM((_NPS,), jnp.float32),
        pltpu.VMEM_SHARED((_NP,), jnp.float32),
        pltpu.SemaphoreType.DMA,
    ),
)
def _sc_hist(src_hbm, dst_hbm, degout_hbm, degin_hbm,
             idx_a, idx_b, ones_v, zero_v, acc_sh, sem):
    c = lax.axis_index("c")
    s = lax.axis_index("s")

    def fill(i, carry):
        zero_v[pl.ds(i * 16, 16)] = jnp.zeros((16,), jnp.float32)
        return carry

    lax.fori_loop(0, _NPS // 16, fill, 0)
    for j in range(8):
        ones_v[pl.ds(j * 16, 16)] = jnp.ones((16,), jnp.float32)
    pltpu.sync_copy(zero_v, acc_sh.at[pl.ds(s * _NPS, _NPS)])
    plsc.subcore_barrier()

    def run(idx_hbm):
        def load(t, idx_v):
            pltpu.sync_copy(idx_hbm.at[pl.ds(s * _ERPT + t * _ECH, _ECH), :],
                            idx_v)

        def scats(idx_v):
            for j in range(_ECH):
                pltpu.async_copy(ones_v, acc_sh.at[idx_v.at[j]], sem,
                                 add=True)

        def wait_scats(idx_v):
            for j in range(_ECH):
                pltpu.make_async_copy(ones_v, acc_sh.at[idx_v.at[j]],
                                      sem).wait()

        load(0, idx_a)

        def pair(u, carry):
            t = u * 2
            scats(idx_a)
            load(t + 1, idx_b)
            wait_scats(idx_a)
            scats(idx_b)
            load(t + 2, idx_a)
            wait_scats(idx_b)
            return carry

        lax.fori_loop(0, (_ECHUNKS - 1) // 2, pair, 0)
        scats(idx_a)
        wait_scats(idx_a)

    @pl.when(c == 0)
    def _():
        run(src_hbm)

    @pl.when(c == 1)
    def _():
        run(dst_hbm)

    plsc.subcore_barrier()
    sl = pl.ds(s * _NPS, _NPS)
    pltpu.sync_copy(acc_sh.at[sl], zero_v)

    @pl.when(c == 0)
    def _():
        pltpu.sync_copy(zero_v, degout_hbm.at[sl])

    @pl.when(c == 1)
    def _():
        pltpu.sync_copy(zero_v, degin_hbm.at[sl])


@functools.partial(
    pl.kernel,
    mesh=_mesh,
    out_type=tuple(
        jax.ShapeDtypeStruct((_NP, 32), jnp.bfloat16) for _ in range(2)
    ),
    compiler_params=pltpu.CompilerParams(use_tc_tiling_on_sc=False,
                                         needs_layout_passes=False),
    scratch_types=(
        pltpu.VMEM((_BCH * _F,), jnp.int32),
        pltpu.VMEM((_BCH * _F,), jnp.int32),
        pltpu.VMEM((_BCH * _F, _D), jnp.float32),
        pltpu.VMEM((_BCH * _F, _D), jnp.float32),
        pltpu.VMEM((_BCH, 32), jnp.float32),
        pltpu.VMEM((_BCH, 32), jnp.float32),
        pltpu.VMEM((_BCH, 32), jnp.bfloat16),
        pltpu.VMEM((_BCH, 32), jnp.bfloat16),
        pltpu.SemaphoreType.DMA,
    ),
)
def _sc_bag(fi_hbm, w_hbm, ep_hbm, ylo_hbm, yhi_hbm,
            idx_a, idx_b, rows_a, rows_b, w_a, w_b,
            ylo_v, yhi_v, sem):
    c = lax.axis_index("c")
    s = lax.axis_index("s")
    wid = s * 2 + c
    node0 = wid * _NPT

    def load(t, idx_v, w_v):
        n0 = node0 + t * _BCH
        pltpu.sync_copy(fi_hbm.at[pl.ds(n0 * _F, _BCH * _F)], idx_v)
        pltpu.sync_copy(w_hbm.at[pl.ds(n0, _BCH), :], w_v)

    def gathers(idx_v, rows_v):
        for j in range(_BST):
            pltpu.async_copy(ep_hbm.at[idx_v.at[pl.ds(j * 128, 128)]],
                             rows_v.at[pl.ds(j * 128, 128), :], sem)

    def wait_gathers(idx_v, rows_v):
        for j in range(_BST):
            pltpu.make_async_copy(ep_hbm.at[idx_v.at[pl.ds(j * 128, 128)]],
                                  rows_v.at[pl.ds(j * 128, 128), :],
                                  sem).wait()

    def compute_store(t, rows_v, w_v):
        def node(i, inner_carry):
            accs = [jnp.zeros((16,), jnp.float32) for _ in range(4)]
            row = i * _F
            wlo = w_v[i, pl.ds(0, 16)]
            whi = w_v[i, pl.ds(16, 16)]
            for q in range(_F):
                wq = wlo[q] if q < 16 else whi[q - 16]
                for k in range(4):
                    accs[k] = accs[k] + wq * rows_v[row + q, pl.ds(k * 16, 16)]
            ylo_v[i, :] = plsc.pack(accs[0], accs[1],
                                    format=plsc.PackFormat.INTERLEAVED)
            yhi_v[i, :] = plsc.pack(accs[2], accs[3],
                                    format=plsc.PackFormat.INTERLEAVED)
            return inner_carry

        lax.fori_loop(0, _BCH, node, 0)
        n0 = node0 + t * _BCH
        pltpu.sync_copy(ylo_v, ylo_hbm.at[pl.ds(n0, _BCH), :])
        pltpu.sync_copy(yhi_v, yhi_hbm.at[pl.ds(n0, _BCH), :])

    load(0, idx_a, w_a)
    gathers(idx_a, rows_a)

    def pair(u, carry):
        t = u * 2
        load(t + 1, idx_b, w_b)
        gathers(idx_b, rows_b)
        wait_gathers(idx_a, rows_a)
        compute_store(t, rows_a, w_a)
        load(t + 2, idx_a, w_a)
        gathers(idx_a, rows_a)
        wait_gathers(idx_b, rows_b)
        compute_store(t + 1, rows_b, w_b)
        return carry

    lax.fori_loop(0, (_BCHUNKS - 1) // 2, pair, 0)
    wait_gathers(idx_a, rows_a)
    compute_store(_BCHUNKS - 1, rows_a, w_a)


@functools.partial(
    pl.kernel,
    mesh=_mesh,
    out_type=jax.ShapeDtypeStruct((_NP, _D), jnp.bfloat16),
    compiler_params=pltpu.CompilerParams(use_tc_tiling_on_sc=False,
                                         needs_layout_passes=False),
    scratch_types=(
        pltpu.VMEM((_ECH, 128), jnp.int32),
        pltpu.VMEM((_ECH, 128), jnp.int32),
        pltpu.VMEM((_ECH, 128), jnp.int32),
        pltpu.VMEM((_ECH, 128), jnp.int32),
        pltpu.VMEM((_ECH * 128, 32), jnp.bfloat16),
        pltpu.VMEM((_ECH * 128, 32), jnp.bfloat16),
        pltpu.VMEM((_ERPT, 32), jnp.bfloat16),
        pltpu.VMEM_SHARED((_NP, 32), jnp.bfloat16),
        pltpu.SemaphoreType.DMA,
        pltpu.SemaphoreType.DMA,
    ),
)
def _sc_edge(src_hbm, dst_hbm, ylo_hbm, yhi_hbm, agg_hbm,
             isrc_a, idst_a, isrc_b, idst_b, rows_a, rows_b,
             zero_v, acc_sh, gsem, ssem):
    c = lax.axis_index("c")
    s = lax.axis_index("s")

    def fillz(i, carry):
        zero_v[i, :] = jnp.zeros((32,), jnp.bfloat16)
        return carry

    def zero_acc():
        lax.fori_loop(0, _ERPT, fillz, 0)
        for q in range(_NPS // _ERPT):
            pltpu.sync_copy(zero_v,
                            acc_sh.at[pl.ds(s * _NPS + q * _ERPT, _ERPT), :])

    def run(y_hbm):
        def load_idx(t, isrc, idst):
            r0 = s * _ERPT + t * _ECH
            pltpu.sync_copy(src_hbm.at[pl.ds(r0, _ECH), :], isrc)
            pltpu.sync_copy(dst_hbm.at[pl.ds(r0, _ECH), :], idst)

        def gathers(isrc, rows):
            for j in range(_ECH):
                pltpu.async_copy(y_hbm.at[isrc.at[j]],
                                 rows.at[pl.ds(j * 128, 128), :], gsem)

        def wait_gathers(isrc, rows):
            for j in range(_ECH):
                pltpu.make_async_copy(y_hbm.at[isrc.at[j]],
                                      rows.at[pl.ds(j * 128, 128), :],
                                      gsem).wait()

        def scatters(idst, rows):
            for j in range(_ECH):
                pltpu.async_copy(rows.at[pl.ds(j * 128, 128), :],
                                 acc_sh.at[idst.at[j]], ssem, add=True)

        def wait_scatters(idst, rows):
            for j in range(_ECH):
                pltpu.make_async_copy(rows.at[pl.ds(j * 128, 128), :],
                                      acc_sh.at[idst.at[j]], ssem).wait()

        load_idx(0, isrc_a, idst_a)
        gathers(isrc_a, rows_a)

        def pair(u, carry):
            t = u * 2
            load_idx(t + 1, isrc_b, idst_b)
            wait_gathers(isrc_a, rows_a)
            scatters(idst_a, rows_a)
            gathers(isrc_b, rows_b)
            wait_scatters(idst_a, rows_a)
            load_idx(t + 2, isrc_a, idst_a)
            gathers(isrc_a, rows_a)
            wait_gathers(isrc_b, rows_b)
            scatters(idst_b, rows_b)
            wait_scatters(idst_b, rows_b)
            return carry

        lax.fori_loop(0, (_ECHUNKS - 1) // 2, pair, 0)
        wait_gathers(isrc_a, rows_a)
        scatters(idst_a, rows_a)
        wait_scatters(idst_a, rows_a)

    def writeout():
        col0 = c * 32

        def wchunk(q, carry):
            sl = pl.ds(s * _NPS + q * _ERPT, _ERPT)
            pltpu.sync_copy(acc_sh.at[sl, :], zero_v)
            pltpu.sync_copy(zero_v, agg_hbm.at[sl, pl.ds(col0, 32)])
            return carry

        lax.fori_loop(0, _NPS // _ERPT, wchunk, 0)

    zero_acc()
    plsc.subcore_barrier()

    @pl.when(c == 0)
    def _():
        run(ylo_hbm)

    @pl.when(c == 1)
    def _():
        run(yhi_hbm)

    plsc.subcore_barrier()
    writeout()


def _tc_w_body(a_ref, s_ref, d_ref, o_ref):
    scale = lax.rsqrt(jnp.maximum(d_ref[...], 1.0))
    o_ref[...] = jnp.dot(a_ref[...], s_ref[...],
                         preferred_element_type=jnp.float32) * scale


def _tc_w(adjs2, smat, deg):
    bn = _NP // 32
    return pl.pallas_call(
        _tc_w_body,
        grid=(32,),
        in_specs=[
            pl.BlockSpec((bn, _F * _F), lambda i: (i, 0)),
            pl.BlockSpec((_F * _F, 32), lambda i: (0, 0)),
            pl.BlockSpec((bn, 1), lambda i: (i, 0)),
        ],
        out_specs=pl.BlockSpec((bn, 32), lambda i: (i, 0)),
        out_shape=jax.ShapeDtypeStruct((_NP, 32), jnp.float32),
    )(adjs2, smat, deg)


def _tc_final_body(a_ref, wg_ref, d_ref, b_ref, o_ref):
    sc = lax.rsqrt(jnp.maximum(d_ref[...], 1.0))
    o_ref[...] = (
        jnp.dot(a_ref[...].astype(jnp.float32), wg_ref[...],
                preferred_element_type=jnp.float32) * sc
        + b_ref[...]
    )


def _tc_final(agg, wg, deg, b2):
    bn = 2000
    return pl.pallas_call(
        _tc_final_body,
        grid=(_N // bn,),
        in_specs=[
            pl.BlockSpec((bn, _D), lambda i: (i, 0)),
            pl.BlockSpec((_D, _D), lambda i: (0, 0)),
            pl.BlockSpec((bn, 1), lambda i: (i, 0)),
            pl.BlockSpec((1, _D), lambda i: (0, 0)),
        ],
        out_specs=pl.BlockSpec((bn, _D), lambda i: (i, 0)),
        out_shape=jax.ShapeDtypeStruct((_N, _D), jnp.float32),
    )(agg, wg, deg, b2)


def kernel(field_embedding, field_adjs, W_gcn, b_gcn, field_index, edges):
    f32 = jnp.float32
    adjs2 = jnp.swapaxes(field_adjs, 1, 2).reshape(_N, _F * _F)
    smat = jnp.pad(jnp.repeat(jnp.eye(_F, dtype=f32), _F, axis=0) / _F,
                   ((0, 0), (0, 32 - _F)))
    n_fi_pad = (_NP - _N) * _F
    fi_pad = (jnp.arange(n_fi_pad, dtype=jnp.int32) * 7919) % _V
    fi2 = jnp.concatenate([field_index.reshape(-1), fi_pad])
    pad_idx = _N + (jnp.arange(_EP - _E, dtype=jnp.int32) % (_NP - _N))
    src2 = jnp.concatenate([edges[0], pad_idx]).reshape(_EROWS, 128)
    dst2 = jnp.concatenate([edges[1], pad_idx]).reshape(_EROWS, 128)

    deg_out, deg_in = _sc_hist(src2, dst2)
    w = _tc_w(adjs2, smat, deg_out.reshape(_NP, 1))
    ylo, yhi = _sc_bag(fi2, w, field_embedding)
    agg = _sc_edge(src2, dst2, ylo, yhi)
    perm = jnp.asarray(
        [h * 32 + (i // 2) + 16 * (i % 2) for h in range(2) for i in range(32)],
        dtype=jnp.int32)
    return _tc_final(agg, W_gcn[perm], deg_in.reshape(_NP, 1),
                     b_gcn.reshape(1, _D))


# speedup vs baseline: 1.1151x; 1.1151x over previous
import functools

import jax
import jax.numpy as jnp
from jax import lax
from jax.experimental import pallas as pl
from jax.experimental.pallas import tpu as pltpu
from jax.experimental.pallas import tpu_sc as plsc

_N = 50000
_F = 20
_V = 100000
_D = 64
_E = 800000

_NP = 50176
_NPT = _NP // 32
_NPS = _NP // 16
_BCH = 32
_BST = _BCH * _F // 128
_BCHUNKS = _NPT // _BCH
_FIROWS = _NP * _F // 128
_EP = 802816
_EROWS = _EP // 128
_ERPT = _EROWS // 16
_ECH = 8
_ECHUNKS = _ERPT // _ECH

_mesh = plsc.VectorSubcoreMesh(core_axis_name="c", subcore_axis_name="s")


@functools.partial(
    pl.kernel,
    mesh=_mesh,
    out_type=(
        jax.ShapeDtypeStruct((_NP,), jnp.float32),
        jax.ShapeDtypeStruct((_NP,), jnp.float32),
    ),
    compiler_params=pltpu.CompilerParams(use_tc_tiling_on_sc=False,
                                         needs_layout_passes=False),
    scratch_types=(
        pltpu.VMEM((_ECH, 128), jnp.int32),
        pltpu.VMEM((_ECH, 128), jnp.int32),
        pltpu.VMEM((128,), jnp.float32),
        pltpu.VMEM((_NPS,), jnp.float32),
        pltpu.VMEM_SHARED((_NP,), jnp.float32),
        pltpu.SemaphoreType.DMA,
    ),
)
def _sc_hist(src_hbm, dst_hbm, degout_hbm, degin_hbm,
             idx_a, idx_b, ones_v, zero_v, acc_sh, sem):
    c = lax.axis_index("c")
    s = lax.axis_index("s")

    def fill(i, carry):
        zero_v[pl.ds(i * 16, 16)] = jnp.zeros((16,), jnp.float32)
        return carry

    lax.fori_loop(0, _NPS // 16, fill, 0)
    for j in range(8):
        ones_v[pl.ds(j * 16, 16)] = jnp.ones((16,), jnp.float32)
    pltpu.sync_copy(zero_v, acc_sh.at[pl.ds(s * _NPS, _NPS)])
    plsc.subcore_barrier()

    def run(idx_hbm):
        def load(t, idx_v):
            pltpu.sync_copy(idx_hbm.at[pl.ds(s * _ERPT + t * _ECH, _ECH), :],
                            idx_v)

        def scats(idx_v):
            for j in range(_ECH):
                pltpu.async_copy(ones_v, acc_sh.at[idx_v.at[j]], sem,
                                 add=True)

        def wait_scats(idx_v):
            for j in range(_ECH):
                pltpu.make_async_copy(ones_v, acc_sh.at[idx_v.at[j]],
                                      sem).wait()

        load(0, idx_a)

        def pair(u, carry):
            t = u * 2
            scats(idx_a)
            load(t + 1, idx_b)
            wait_scats(idx_a)
            scats(idx_b)
            load(t + 2, idx_a)
            wait_scats(idx_b)
            return carry

        lax.fori_loop(0, (_ECHUNKS - 1) // 2, pair, 0)
        scats(idx_a)
        wait_scats(idx_a)

    @pl.when(c == 0)
    def _():
        run(src_hbm)

    @pl.when(c == 1)
    def _():
        run(dst_hbm)

    plsc.subcore_barrier()
    sl = pl.ds(s * _NPS, _NPS)
    pltpu.sync_copy(acc_sh.at[sl], zero_v)

    @pl.when(c == 0)
    def _():
        pltpu.sync_copy(zero_v, degout_hbm.at[sl])

    @pl.when(c == 1)
    def _():
        pltpu.sync_copy(zero_v, degin_hbm.at[sl])


@functools.partial(
    pl.kernel,
    mesh=_mesh,
    out_type=tuple(
        jax.ShapeDtypeStruct((_NP, 32), jnp.bfloat16) for _ in range(2)
    ),
    compiler_params=pltpu.CompilerParams(use_tc_tiling_on_sc=False,
                                         needs_layout_passes=False),
    scratch_types=(
        pltpu.VMEM((_BCH * _F,), jnp.int32),
        pltpu.VMEM((_BCH * _F,), jnp.int32),
        pltpu.VMEM((_BCH * _F, _D), jnp.float32),
        pltpu.VMEM((_BCH * _F, _D), jnp.float32),
        pltpu.VMEM((_BCH, 32), jnp.float32),
        pltpu.VMEM((_BCH, 32), jnp.float32),
        pltpu.VMEM((_BCH, 32), jnp.bfloat16),
        pltpu.VMEM((_BCH, 32), jnp.bfloat16),
        pltpu.SemaphoreType.DMA,
    ),
)
def _sc_bag(fi_hbm, w_hbm, ep_hbm, ylo_hbm, yhi_hbm,
            idx_a, idx_b, rows_a, rows_b, w_a, w_b,
            ylo_v, yhi_v, sem):
    c = lax.axis_index("c")
    s = lax.axis_index("s")
    wid = s * 2 + c
    node0 = wid * _NPT

    def load(t, idx_v, w_v):
        n0 = node0 + t * _BCH
        pltpu.sync_copy(fi_hbm.at[pl.ds(n0 * _F, _BCH * _F)], idx_v)
        pltpu.sync_copy(w_hbm.at[pl.ds(n0, _BCH), :], w_v)

    def gathers(idx_v, rows_v):
        for j in range(_BST):
            pltpu.async_copy(ep_hbm.at[idx_v.at[pl.ds(j * 128, 128)]],
                             rows_v.at[pl.ds(j * 128, 128), :], sem)

    def wait_gathers(idx_v, rows_v):
        for j in range(_BST):
            pltpu.make_async_copy(ep_hbm.at[idx_v.at[pl.ds(j * 128, 128)]],
                                  rows_v.at[pl.ds(j * 128, 128), :],
                                  sem).wait()

    def compute_store(t, rows_v, w_v):
        def node(i, inner_carry):
            accs = [jnp.zeros((16,), jnp.float32) for _ in range(4)]
            row = i * _F
            wlo = w_v[i, pl.ds(0, 16)]
            whi = w_v[i, pl.ds(16, 16)]
            for q in range(_F):
                wq = wlo[q] if q < 16 else whi[q - 16]
                for k in range(4):
                    accs[k] = accs[k] + wq * rows_v[row + q, pl.ds(k * 16, 16)]
            ylo_v[i, :] = plsc.pack(accs[0], accs[1],
                                    format=plsc.PackFormat.INTERLEAVED)
            yhi_v[i, :] = plsc.pack(accs[2], accs[3],
                                    format=plsc.PackFormat.INTERLEAVED)
            return inner_carry

        lax.fori_loop(0, _BCH, node, 0)
        n0 = node0 + t * _BCH
        pltpu.sync_copy(ylo_v, ylo_hbm.at[pl.ds(n0, _BCH), :])
        pltpu.sync_copy(yhi_v, yhi_hbm.at[pl.ds(n0, _BCH), :])

    load(0, idx_a, w_a)
    gathers(idx_a, rows_a)

    def pair(u, carry):
        t = u * 2
        load(t + 1, idx_b, w_b)
        gathers(idx_b, rows_b)
        wait_gathers(idx_a, rows_a)
        compute_store(t, rows_a, w_a)
        load(t + 2, idx_a, w_a)
        gathers(idx_a, rows_a)
        wait_gathers(idx_b, rows_b)
        compute_store(t + 1, rows_b, w_b)
        return carry

    lax.fori_loop(0, (_BCHUNKS - 1) // 2, pair, 0)
    wait_gathers(idx_a, rows_a)
    compute_store(_BCHUNKS - 1, rows_a, w_a)


@functools.partial(
    pl.kernel,
    mesh=_mesh,
    out_type=jax.ShapeDtypeStruct((_NP, _D), jnp.bfloat16),
    compiler_params=pltpu.CompilerParams(use_tc_tiling_on_sc=False,
                                         needs_layout_passes=False),
    scratch_types=(
        pltpu.VMEM((_ECH, 128), jnp.int32),
        pltpu.VMEM((_ECH, 128), jnp.int32),
        pltpu.VMEM((_ECH, 128), jnp.int32),
        pltpu.VMEM((_ECH, 128), jnp.int32),
        pltpu.VMEM((_ECH * 128, 32), jnp.bfloat16),
        pltpu.VMEM((_ECH * 128, 32), jnp.bfloat16),
        pltpu.VMEM((_ERPT, 32), jnp.bfloat16),
        pltpu.VMEM_SHARED((_NP, 32), jnp.bfloat16),
        pltpu.SemaphoreType.DMA,
        pltpu.SemaphoreType.DMA,
    ),
)
def _sc_edge(src_hbm, dst_hbm, ylo_hbm, yhi_hbm, agg_hbm,
             isrc_a, idst_a, isrc_b, idst_b, rows_a, rows_b,
             zero_v, acc_sh, gsem, ssem):
    c = lax.axis_index("c")
    s = lax.axis_index("s")

    def fillz(i, carry):
        zero_v[i, :] = jnp.zeros((32,), jnp.bfloat16)
        return carry

    def zero_acc():
        lax.fori_loop(0, _ERPT, fillz, 0)
        for q in range(_NPS // _ERPT):
            pltpu.sync_copy(zero_v,
                            acc_sh.at[pl.ds(s * _NPS + q * _ERPT, _ERPT), :])

    def run(y_hbm):
        def load_idx(t, isrc, idst):
            r0 = s * _ERPT + t * _ECH
            pltpu.sync_copy(src_hbm.at[pl.ds(r0, _ECH), :], isrc)
            pltpu.sync_copy(dst_hbm.at[pl.ds(r0, _ECH), :], idst)

        def gathers(isrc, rows):
            for j in range(_ECH):
                pltpu.async_copy(y_hbm.at[isrc.at[j]],
                                 rows.at[pl.ds(j * 128, 128), :], gsem)

        def wait_gathers(isrc, rows):
            for j in range(_ECH):
                pltpu.make_async_copy(y_hbm.at[isrc.at[j]],
                                      rows.at[pl.ds(j * 128, 128), :],
                                      gsem).wait()

        def scatters(idst, rows):
            for j in range(_ECH):
                pltpu.async_copy(rows.at[pl.ds(j * 128, 128), :],
                                 acc_sh.at[idst.at[j]], ssem, add=True)

        def wait_scatters(idst, rows):
            for j in range(_ECH):
                pltpu.make_async_copy(rows.at[pl.ds(j * 128, 128), :],
                                      acc_sh.at[idst.at[j]], ssem).wait()

        load_idx(0, isrc_a, idst_a)
        gathers(isrc_a, rows_a)

        def pair(u, carry):
            t = u * 2
            load_idx(t + 1, isrc_b, idst_b)
            wait_gathers(isrc_a, rows_a)
            scatters(idst_a, rows_a)
            gathers(isrc_b, rows_b)
            wait_scatters(idst_a, rows_a)
            load_idx(t + 2, isrc_a, idst_a)
            gathers(isrc_a, rows_a)
            wait_gathers(isrc_b, rows_b)
            scatters(idst_b, rows_b)
            wait_scatters(idst_b, rows_b)
            return carry

        lax.fori_loop(0, (_ECHUNKS - 1) // 2, pair, 0)
        wait_gathers(isrc_a, rows_a)
        scatters(idst_a, rows_a)
        wait_scatters(idst_a, rows_a)

    def writeout():
        col0 = c * 32

        def wchunk(q, carry):
            sl = pl.ds(s * _NPS + q * _ERPT, _ERPT)
            pltpu.sync_copy(acc_sh.at[sl, :], zero_v)
            pltpu.sync_copy(zero_v, agg_hbm.at[sl, pl.ds(col0, 32)])
            return carry

        lax.fori_loop(0, _NPS // _ERPT, wchunk, 0)

    zero_acc()
    plsc.subcore_barrier()

    @pl.when(c == 0)
    def _():
        run(ylo_hbm)

    @pl.when(c == 1)
    def _():
        run(yhi_hbm)

    plsc.subcore_barrier()
    writeout()


def _tc_w_body(a_ref, s_ref, d_ref, o_ref):
    scale = lax.rsqrt(jnp.maximum(d_ref[...], 1.0))
    o_ref[...] = jnp.dot(a_ref[...], s_ref[...],
                         preferred_element_type=jnp.float32) * scale


def _tc_w(adjs2, smat, deg):
    bn = _NP // 32
    return pl.pallas_call(
        _tc_w_body,
        grid=(32,),
        in_specs=[
            pl.BlockSpec((bn, _F * _F), lambda i: (i, 0)),
            pl.BlockSpec((_F * _F, 32), lambda i: (0, 0)),
            pl.BlockSpec((bn, 1), lambda i: (i, 0)),
        ],
        out_specs=pl.BlockSpec((bn, 32), lambda i: (i, 0)),
        out_shape=jax.ShapeDtypeStruct((_NP, 32), jnp.float32),
    )(adjs2, smat, deg)


def _tc_final_body(a_ref, wg_ref, d_ref, b_ref, o_ref):
    sc = lax.rsqrt(jnp.maximum(d_ref[...], 1.0))
    o_ref[...] = (
        jnp.dot(a_ref[...].astype(jnp.float32), wg_ref[...],
                preferred_element_type=jnp.float32) * sc
        + b_ref[...]
    )


def _tc_final(agg, wg, deg, b2):
    bn = 2000
    return pl.pallas_call(
        _tc_final_body,
        grid=(_N // bn,),
        in_specs=[
            pl.BlockSpec((bn, _D), lambda i: (i, 0)),
            pl.BlockSpec((_D, _D), lambda i: (0, 0)),
            pl.BlockSpec((bn, 1), lambda i: (i, 0)),
            pl.BlockSpec((1, _D), lambda i: (0, 0)),
        ],
        out_specs=pl.BlockSpec((bn, _D), lambda i: (i, 0)),
        out_shape=jax.ShapeDtypeStruct((_N, _D), jnp.float32),
    )(agg, wg, deg, b2)


def kernel(field_embedding, field_adjs, W_gcn, b_gcn, field_index, edges):
    f32 = jnp.float32
    adjs2 = field_adjs.reshape(_N, _F * _F)
    smat = jnp.pad(jnp.tile(jnp.eye(_F, dtype=f32), (_F, 1)) / _F,
                   ((0, 0), (0, 32 - _F)))
    n_fi_pad = (_NP - _N) * _F
    fi_pad = (jnp.arange(n_fi_pad, dtype=jnp.int32) * 7919) % _V
    fi2 = jnp.concatenate([field_index.reshape(-1), fi_pad])
    pad_idx = _N + (jnp.arange(_EP - _E, dtype=jnp.int32) % (_NP - _N))
    src2 = jnp.concatenate([edges[0], pad_idx]).reshape(_EROWS, 128)
    dst2 = jnp.concatenate([edges[1], pad_idx]).reshape(_EROWS, 128)

    deg_out, deg_in = _sc_hist(src2, dst2)
    w = _tc_w(adjs2, smat, deg_out.reshape(_NP, 1))
    ylo, yhi = _sc_bag(fi2, w, field_embedding)
    agg = _sc_edge(src2, dst2, ylo, yhi)
    perm = jnp.asarray(
        [h * 32 + (i // 2) + 16 * (i % 2) for h in range(2) for i in range(32)],
        dtype=jnp.int32)
    return _tc_final(agg, W_gcn[perm], deg_in.reshape(_NP, 1),
                     b_gcn.reshape(1, _D))
